# baseline (speedup 1.0000x reference)
import functools

import jax
import jax.numpy as jnp
from jax import lax
from jax.experimental import pallas as pl
from jax.experimental.pallas import tpu as pltpu
from jax.experimental.pallas import tpu_sc as plsc

N = 10000
NL = 2 * N
NU = 2 * N
E4 = 4 * 160000
CHUNK = 128
IDXROWS = -(-E4 // (256 * CHUNK)) * 256
EPAD = IDXROWS * CHUNK
TRASH = NU
TBL_ROWS = NU + 8
ACC_ROWS = 20480
PH3 = 6720
ACC3 = 6784


NB = 32


@functools.lru_cache(maxsize=None)
def _make_edge_aggregate(width, shard_by_core, npass, acc_rows):
    nshard = 32 if shard_by_core else 16
    nrows = IDXROWS // nshard
    zr = acc_rows // 16
    mesh = plsc.VectorSubcoreMesh(core_axis_name="c", subcore_axis_name="s")

    @functools.partial(
        pl.kernel,
        out_type=jax.ShapeDtypeStruct((2, npass, acc_rows, width), jnp.float32),
        mesh=mesh,
        scratch_types=[
            pltpu.VMEM((nrows, CHUNK), jnp.int32),
            pltpu.VMEM((nrows, CHUNK), jnp.int32),
            pltpu.VMEM((CHUNK, width), jnp.float32),
            pltpu.VMEM((CHUNK, width), jnp.float32),
            pltpu.VMEM_SHARED((acc_rows, width), jnp.float32),
            pltpu.SemaphoreType.DMA,
        ],
        compiler_params=pltpu.CompilerParams(use_tc_tiling_on_sc=False),
    )
    def k(ta, tb, si_h, di_h, z_h, out, sv, dv, r0, r1, acc, s0):
        c = lax.axis_index("c")
        t = lax.axis_index("s")
        base = (t * 2 + c) * nrows if shard_by_core else t * nrows
        pltpu.sync_copy(si_h.at[pl.ds(base, nrows)], sv)

        for p in range(npass):
            pltpu.sync_copy(z_h, acc.at[pl.ds(t * zr, zr)])
            plsc.subcore_barrier()

            def run(tbl):
                pltpu.sync_copy(di_h.at[p].at[pl.ds(base, nrows)], dv)

                def body(i, carry2):
                    j = 2 * i
                    d0 = pltpu.async_copy(tbl.at[sv.at[j]], r0, s0)
                    d1 = pltpu.async_copy(tbl.at[sv.at[j + 1]], r1, s0)
                    d0.wait()
                    d1.wait()
                    pltpu.sync_copy(r0, acc.at[dv.at[j]], add=True)
                    pltpu.sync_copy(r1, acc.at[dv.at[j + 1]], add=True)
                    return carry2

                lax.fori_loop(0, nrows // 2, body, 0)

            @pl.when(c == 0)
            def _():
                run(ta)

            @pl.when(c == 1)
            def _():
                run(tb)

            plsc.subcore_barrier()
            pltpu.sync_copy(acc.at[pl.ds(t * zr, zr)],
                            out.at[c, p, pl.ds(t * zr, zr)])
            plsc.subcore_barrier()

    return k


def _edge_aggregate(tbl_a, tbl_b, sidx, didx, zrows, *, width, shard_by_core):
    npass = didx.shape[0]
    acc_rows = zrows.shape[0] * 16
    return _make_edge_aggregate(width, shard_by_core, npass, acc_rows)(
        tbl_a, tbl_b, sidx, didx, zrows)


_BLK = 1000


def _lkk_body(hL_ref, hU_ref, agg_ref, dis_ref, W_ref, b_ref, oL_ref, oU_ref):
    W = W_ref[...]
    b = b_ref[0:1, :]
    dis = dis_ref[...]
    m = dis * agg_ref[...] + dis * dis * hU_ref[...]
    zL = lax.dot_general(hL_ref[...], W, (((1,), (1,)), ((), ())),
                         preferred_element_type=jnp.float32)
    zU = lax.dot_general(m, W, (((1,), (1,)), ((), ())),
                         preferred_element_type=jnp.float32)
    oL_ref[...] = jnp.maximum(zL + b, 0.0)
    oU_ref[...] = jnp.maximum(zU + b, 0.0)


def _layer(hL, hU, agg, disb, W, b8):
    grid = NL // _BLK
    return pl.pallas_call(
        _lkk_body,
        grid=(grid,),
        in_specs=[
            pl.BlockSpec((_BLK, 128), lambda i: (i, 0)),
            pl.BlockSpec((_BLK, 128), lambda i: (i, 0)),
            pl.BlockSpec((_BLK, 128), lambda i: (i, 0)),
            pl.BlockSpec((_BLK, 128), lambda i: (i, 0)),
            pl.BlockSpec((128, 128), lambda i: (0, 0)),
            pl.BlockSpec((8, 128), lambda i: (0, 0)),
        ],
        out_specs=[
            pl.BlockSpec((_BLK, 128), lambda i: (i, 0)),
            pl.BlockSpec((_BLK, 128), lambda i: (i, 0)),
        ],
        out_shape=[
            jax.ShapeDtypeStruct((NL, 128), jnp.float32),
            jax.ShapeDtypeStruct((NL, 128), jnp.float32),
        ],
    )(hL, hU, agg, disb, W, b8)


def _pool_body(hL_ref, hU_ref, o_ref):
    g = pl.program_id(0)

    @pl.when(g == 0)
    def _():
        o_ref[...] = jnp.zeros_like(o_ref)

    sL = jnp.sum(hL_ref[...], axis=0, keepdims=True)
    sU = jnp.sum(hU_ref[...], axis=0, keepdims=True)
    riota = lax.broadcasted_iota(jnp.int32, (8, 128), 0)
    rowL = jnp.where(g < 10, 0, 1)
    rowU = jnp.where(g < 10, 2, 3)
    upd = jnp.where(riota == rowL, sL, 0.0) + jnp.where(riota == rowU, sU, 0.0)
    o_ref[...] += upd


def _pool(hL, hU):
    grid = NL // _BLK
    return pl.pallas_call(
        _pool_body,
        grid=(grid,),
        in_specs=[
            pl.BlockSpec((_BLK, 128), lambda i: (i, 0)),
            pl.BlockSpec((_BLK, 128), lambda i: (i, 0)),
        ],
        out_specs=pl.BlockSpec((8, 128), lambda i: (0, 0)),
        out_shape=jax.ShapeDtypeStruct((8, 128), jnp.float32),
    )(hL, hU)


NOPAD = 10240
_OBLK = 1024


def _fin_body(p_ref, m1_ref, b1_ref, m2_ref, b2_ref, ow_ref, ob_ref, o_ref):
    z = p_ref[...]
    z = jnp.maximum(lax.dot_general(z, m1_ref[...], (((1,), (1,)), ((), ())),
                                    preferred_element_type=jnp.float32)
                    + b1_ref[0:1, :], 0.0)
    z = jnp.maximum(lax.dot_general(z, m2_ref[...], (((1,), (1,)), ((), ())),
                                    preferred_element_type=jnp.float32)
                    + b2_ref[0:1, :], 0.0)
    logits = lax.dot_general(z, ow_ref[...], (((1,), (1,)), ((), ())),
                             preferred_element_type=jnp.float32) + ob_ref[...]
    o_ref[...] = jax.nn.sigmoid(logits)


def _fin(pool8, M1w, M1b8, M2w, M2b8, OwP, ObP8):
    grid = NOPAD // _OBLK
    return pl.pallas_call(
        _fin_body,
        grid=(grid,),
        in_specs=[
            pl.BlockSpec((8, 128), lambda i: (0, 0)),
            pl.BlockSpec((128, 128), lambda i: (0, 0)),
            pl.BlockSpec((8, 128), lambda i: (0, 0)),
            pl.BlockSpec((128, 128), lambda i: (0, 0)),
            pl.BlockSpec((8, 128), lambda i: (0, 0)),
            pl.BlockSpec((_OBLK, 128), lambda i: (i, 0)),
            pl.BlockSpec((8, _OBLK), lambda i: (0, i)),
        ],
        out_specs=pl.BlockSpec((8, _OBLK), lambda i: (0, i)),
        out_shape=jax.ShapeDtypeStruct((8, NOPAD), jnp.float32),
    )(pool8, M1w, M1b8, M2w, M2b8, OwP, ObP8)


def _b8(v):
    return jnp.broadcast_to(v[None, :], (8, v.shape[0])).astype(jnp.float32)


def _padcols(a, width=128):
    return jnp.pad(a, ((0, 0), (0, width - a.shape[1])))


def kernel(actions, node_features, edge_index, W1, b1, W2, b2, W3, b3,
           M1w, M1b, M2w, M2b, Ow, Ob):
    B_ = actions.shape[0]
    nf = node_features.astype(jnp.float32).reshape(B_, N)
    x = jnp.stack((actions[:, :, 0], actions[:, :, 1], nf), axis=2)
    x = x.reshape(B_ * N, 3)
    xL, xU = x[:NL], x[NL:]

    ei = edge_index + (jnp.arange(B_, dtype=edge_index.dtype) * N)[:, None, None]
    ei = ei.reshape(2, E4)
    pad = jnp.full((EPAD - E4,), TRASH, dtype=jnp.int32)
    sidx = jnp.concatenate([ei[0].astype(jnp.int32), pad]).reshape(IDXROWS, CHUNK)
    dfull = jnp.concatenate([(ei[1] - NL).astype(jnp.int32), pad])
    didx = dfull.reshape(1, IDXROWS, CHUNK)
    dps = []
    for p in range(3):
        inp = (dfull >= p * PH3) & (dfull < (p + 1) * PH3)
        dps.append(jnp.where(inp, dfull - p * PH3, PH3))
    didx3 = jnp.stack(dps).reshape(3, IDXROWS, CHUNK)

    xtbl = jnp.zeros((TBL_ROWS, 16), jnp.float32)
    xtbl = xtbl.at[:NL, :3].set(xL).at[:NL, 3].set(1.0)
    z16 = jnp.zeros((ACC_ROWS // 16, 16), jnp.float32)
    g16 = _edge_aggregate(xtbl, xtbl, sidx, didx, z16,
                          width=16, shard_by_core=True)
    g16s = g16[0, 0, :NU] + g16[1, 0, :NU]
    cnt = g16s[:, 3]
    dis = lax.rsqrt(1.0 + cnt)
    disb = jnp.broadcast_to(dis[:, None], (NU, 128))
    agg1 = _padcols(g16s[:, :3])

    w1p = jnp.zeros((128, 128), jnp.float32).at[:, :3].set(W1)
    hL, hU = _layer(_padcols(xL), _padcols(xU), agg1, disb, w1p, _b8(b1))

    z64 = jnp.zeros((ACC3 // 16, 64), jnp.float32)

    def body(carry, wb):
        hLc, hUc = carry
        W, b8 = wb
        tbl = jnp.zeros((TBL_ROWS, 128), jnp.float32).at[:NL].set(hLc)
        g = _edge_aggregate(tbl[:, :64], tbl[:, 64:], sidx, didx3, z64,
                            width=64, shard_by_core=False)
        agg = jnp.concatenate(
            [jnp.concatenate([g[c, 0, :PH3], g[c, 1, :PH3],
                              g[c, 2, :NU - 2 * PH3]], axis=0)
             for c in range(2)], axis=1)
        hLn, hUn = _layer(hLc, hUc, agg, disb, W, b8)
        return (hLn, hUn), 0

    (hL, hU), _ = lax.scan(
        body, (hL, hU),
        (jnp.stack([W2, W3]), jnp.stack([_b8(b2), _b8(b3)])))

    pool8 = _pool(hL, hU)

    OwP = jnp.zeros((NOPAD, 128), jnp.float32).at[:N].set(Ow)
    ObP8 = _b8(jnp.pad(Ob, (0, NOPAD - N)))
    out8 = _fin(pool8, M1w, _b8(M1b), M2w, _b8(M2b), OwP, ObP8)
    return out8[:B_, :N]


# speedup vs baseline: 2.3253x; 2.3253x over previous
import functools

import jax
import jax.numpy as jnp
from jax import lax
from jax.experimental import pallas as pl
from jax.experimental.pallas import tpu as pltpu
from jax.experimental.pallas import tpu_sc as plsc

N = 10000
NL = 2 * N
NU = 2 * N
E4 = 4 * 160000
CHUNK = 128
IDXROWS = -(-E4 // (256 * CHUNK)) * 256
EPAD = IDXROWS * CHUNK
TRASH = NU
TBL_ROWS = NU + 8
ACC_ROWS = 20480
PH3 = 6720
ACC3 = 6784


NB = 32


@functools.lru_cache(maxsize=None)
def _make_edge_aggregate(width, shard_by_core, npass, acc_rows):
    nshard = 32 if shard_by_core else 16
    nrows = IDXROWS // nshard
    zr = acc_rows // 16
    mesh = plsc.VectorSubcoreMesh(core_axis_name="c", subcore_axis_name="s")

    @functools.partial(
        pl.kernel,
        out_type=jax.ShapeDtypeStruct((2, npass, acc_rows, width), jnp.float32),
        mesh=mesh,
        scratch_types=[
            pltpu.VMEM((nrows, CHUNK), jnp.int32),
            pltpu.VMEM((nrows, CHUNK), jnp.int32),
            pltpu.VMEM((CHUNK, width), jnp.float32),
            pltpu.VMEM((CHUNK, width), jnp.float32),
            pltpu.VMEM_SHARED((acc_rows, width), jnp.float32),
            pltpu.SemaphoreType.DMA,
        ],
        compiler_params=pltpu.CompilerParams(use_tc_tiling_on_sc=False),
    )
    def k(ta, tb, si_h, di_h, z_h, out, sv, dv, r0, r1, acc, s0):
        c = lax.axis_index("c")
        t = lax.axis_index("s")
        base = (t * 2 + c) * nrows if shard_by_core else t * nrows
        pltpu.sync_copy(si_h.at[pl.ds(base, nrows)], sv)

        for p in range(npass):
            pltpu.sync_copy(z_h, acc.at[pl.ds(t * zr, zr)])
            plsc.subcore_barrier()

            def run(tbl):
                pltpu.sync_copy(di_h.at[p].at[pl.ds(base, nrows)], dv)

                def body(i, carry2):
                    pltpu.async_copy(tbl.at[sv.at[i]], r0, s0).wait()
                    pltpu.sync_copy(r0, acc.at[dv.at[i]], add=True)
                    return carry2

                lax.fori_loop(0, nrows, body, 0)

            @pl.when(c == 0)
            def _():
                run(ta)

            @pl.when(c == 1)
            def _():
                run(tb)

            plsc.subcore_barrier()
            pltpu.sync_copy(acc.at[pl.ds(t * zr, zr)],
                            out.at[c, p, pl.ds(t * zr, zr)])
            plsc.subcore_barrier()

    return k


def _edge_aggregate(tbl_a, tbl_b, sidx, didx, zrows, *, width, shard_by_core):
    npass = didx.shape[0]
    acc_rows = zrows.shape[0] * 16
    return _make_edge_aggregate(width, shard_by_core, npass, acc_rows)(
        tbl_a, tbl_b, sidx, didx, zrows)


_BLK = 1000


def _lkk_body(hL_ref, hU_ref, agg_ref, dis_ref, W_ref, b_ref, oL_ref, oU_ref):
    W = W_ref[...]
    b = b_ref[0:1, :]
    dis = dis_ref[...]
    m = dis * agg_ref[...] + dis * dis * hU_ref[...]
    zL = lax.dot_general(hL_ref[...], W, (((1,), (1,)), ((), ())),
                         preferred_element_type=jnp.float32)
    zU = lax.dot_general(m, W, (((1,), (1,)), ((), ())),
                         preferred_element_type=jnp.float32)
    oL_ref[...] = jnp.maximum(zL + b, 0.0)
    oU_ref[...] = jnp.maximum(zU + b, 0.0)


def _layer(hL, hU, agg, disb, W, b8):
    grid = NL // _BLK
    return pl.pallas_call(
        _lkk_body,
        grid=(grid,),
        in_specs=[
            pl.BlockSpec((_BLK, 128), lambda i: (i, 0)),
            pl.BlockSpec((_BLK, 128), lambda i: (i, 0)),
            pl.BlockSpec((_BLK, 128), lambda i: (i, 0)),
            pl.BlockSpec((_BLK, 128), lambda i: (i, 0)),
            pl.BlockSpec((128, 128), lambda i: (0, 0)),
            pl.BlockSpec((8, 128), lambda i: (0, 0)),
        ],
        out_specs=[
            pl.BlockSpec((_BLK, 128), lambda i: (i, 0)),
            pl.BlockSpec((_BLK, 128), lambda i: (i, 0)),
        ],
        out_shape=[
            jax.ShapeDtypeStruct((NL, 128), jnp.float32),
            jax.ShapeDtypeStruct((NL, 128), jnp.float32),
        ],
    )(hL, hU, agg, disb, W, b8)


def _pool_body(hL_ref, hU_ref, o_ref):
    g = pl.program_id(0)

    @pl.when(g == 0)
    def _():
        o_ref[...] = jnp.zeros_like(o_ref)

    sL = jnp.sum(hL_ref[...], axis=0, keepdims=True)
    sU = jnp.sum(hU_ref[...], axis=0, keepdims=True)
    riota = lax.broadcasted_iota(jnp.int32, (8, 128), 0)
    rowL = jnp.where(g < 10, 0, 1)
    rowU = jnp.where(g < 10, 2, 3)
    upd = jnp.where(riota == rowL, sL, 0.0) + jnp.where(riota == rowU, sU, 0.0)
    o_ref[...] += upd


def _pool(hL, hU):
    grid = NL // _BLK
    return pl.pallas_call(
        _pool_body,
        grid=(grid,),
        in_specs=[
            pl.BlockSpec((_BLK, 128), lambda i: (i, 0)),
            pl.BlockSpec((_BLK, 128), lambda i: (i, 0)),
        ],
        out_specs=pl.BlockSpec((8, 128), lambda i: (0, 0)),
        out_shape=jax.ShapeDtypeStruct((8, 128), jnp.float32),
    )(hL, hU)


NOPAD = 10240
_OBLK = 1024


def _fin_body(p_ref, m1_ref, b1_ref, m2_ref, b2_ref, ow_ref, ob_ref, o_ref):
    z = p_ref[...]
    z = jnp.maximum(lax.dot_general(z, m1_ref[...], (((1,), (1,)), ((), ())),
                                    preferred_element_type=jnp.float32)
                    + b1_ref[0:1, :], 0.0)
    z = jnp.maximum(lax.dot_general(z, m2_ref[...], (((1,), (1,)), ((), ())),
                                    preferred_element_type=jnp.float32)
                    + b2_ref[0:1, :], 0.0)
    logits = lax.dot_general(z, ow_ref[...], (((1,), (1,)), ((), ())),
                             preferred_element_type=jnp.float32) + ob_ref[...]
    o_ref[...] = jax.nn.sigmoid(logits)


def _fin(pool8, M1w, M1b8, M2w, M2b8, OwP, ObP8):
    grid = NOPAD // _OBLK
    return pl.pallas_call(
        _fin_body,
        grid=(grid,),
        in_specs=[
            pl.BlockSpec((8, 128), lambda i: (0, 0)),
            pl.BlockSpec((128, 128), lambda i: (0, 0)),
            pl.BlockSpec((8, 128), lambda i: (0, 0)),
            pl.BlockSpec((128, 128), lambda i: (0, 0)),
            pl.BlockSpec((8, 128), lambda i: (0, 0)),
            pl.BlockSpec((_OBLK, 128), lambda i: (i, 0)),
            pl.BlockSpec((8, _OBLK), lambda i: (0, i)),
        ],
        out_specs=pl.BlockSpec((8, _OBLK), lambda i: (0, i)),
        out_shape=jax.ShapeDtypeStruct((8, NOPAD), jnp.float32),
    )(pool8, M1w, M1b8, M2w, M2b8, OwP, ObP8)


def _b8(v):
    return jnp.broadcast_to(v[None, :], (8, v.shape[0])).astype(jnp.float32)


def _padcols(a, width=128):
    return jnp.pad(a, ((0, 0), (0, width - a.shape[1])))


def kernel(actions, node_features, edge_index, W1, b1, W2, b2, W3, b3,
           M1w, M1b, M2w, M2b, Ow, Ob):
    B_ = actions.shape[0]
    nf = node_features.astype(jnp.float32).reshape(B_, N)
    x = jnp.stack((actions[:, :, 0], actions[:, :, 1], nf), axis=2)
    x = x.reshape(B_ * N, 3)
    xL, xU = x[:NL], x[NL:]

    ei = edge_index + (jnp.arange(B_, dtype=edge_index.dtype) * N)[:, None, None]
    ei = ei.reshape(2, E4)
    pad = jnp.full((EPAD - E4,), TRASH, dtype=jnp.int32)
    sidx = jnp.concatenate([ei[0].astype(jnp.int32), pad]).reshape(IDXROWS, CHUNK)
    dfull = jnp.concatenate([(ei[1] - NL).astype(jnp.int32), pad])
    didx = dfull.reshape(1, IDXROWS, CHUNK)
    dps = []
    for p in range(3):
        inp = (dfull >= p * PH3) & (dfull < (p + 1) * PH3)
        dps.append(jnp.where(inp, dfull - p * PH3, PH3))
    didx3 = jnp.stack(dps).reshape(3, IDXROWS, CHUNK)

    xtbl = jnp.zeros((TBL_ROWS, 16), jnp.float32)
    xtbl = xtbl.at[:NL, :3].set(xL).at[:NL, 3].set(1.0)
    z16 = jnp.zeros((ACC_ROWS // 16, 16), jnp.float32)
    g16 = _edge_aggregate(xtbl, xtbl, sidx, didx, z16,
                          width=16, shard_by_core=True)
    g16s = g16[0, 0, :NU] + g16[1, 0, :NU]
    cnt = g16s[:, 3]
    dis = lax.rsqrt(1.0 + cnt)
    disb = jnp.broadcast_to(dis[:, None], (NU, 128))
    agg1 = _padcols(g16s[:, :3])

    w1p = jnp.zeros((128, 128), jnp.float32).at[:, :3].set(W1)
    hL, hU = _layer(_padcols(xL), _padcols(xU), agg1, disb, w1p, _b8(b1))

    z32 = jnp.zeros((ACC_ROWS // 16, 32), jnp.float32)

    def body(carry, wb):
        hLc, hUc = carry
        W, b8 = wb
        tbl = jnp.zeros((TBL_ROWS, 128), jnp.float32).at[:NL].set(hLc)
        gA = _edge_aggregate(tbl[:, 0:32], tbl[:, 32:64], sidx, didx, z32,
                             width=32, shard_by_core=False)
        gB = _edge_aggregate(tbl[:, 64:96], tbl[:, 96:128], sidx, didx, z32,
                             width=32, shard_by_core=False)
        agg = jnp.concatenate([gA[0, 0, :NU], gA[1, 0, :NU],
                               gB[0, 0, :NU], gB[1, 0, :NU]], axis=1)
        hLn, hUn = _layer(hLc, hUc, agg, disb, W, b8)
        return (hLn, hUn), 0

    (hL, hU), _ = lax.scan(
        body, (hL, hU),
        (jnp.stack([W2, W3]), jnp.stack([_b8(b2), _b8(b3)])))

    pool8 = _pool(hL, hU)

    OwP = jnp.zeros((NOPAD, 128), jnp.float32).at[:N].set(Ow)
    ObP8 = _b8(jnp.pad(Ob, (0, NOPAD - N)))
    out8 = _fin(pool8, M1w, _b8(M1b), M2w, _b8(M2b), OwP, ObP8)
    return out8[:B_, :N]
